# Initial kernel scaffold; baseline (speedup 1.0000x reference)
import jax
import jax.numpy as jnp
from jax.experimental import pallas as pl


def kernel(x, edge_index, batch, W1, b1, W2, b2, W3, b3, Wc, bc):
    raise NotImplementedError("write your pallas kernel here")


# speedup vs baseline: 28.5162x; 28.5162x over previous
import functools

import jax
import jax.numpy as jnp
from jax import lax
from jax.experimental import pallas as pl
from jax.experimental.pallas import tpu as pltpu
from jax.experimental.pallas import tpu_sc as plsc

N = 10000
E = 320000
H = 128
HH = H // 2
C = 10
G = 64

NC = 2
NS = 16
NW = NC * NS

K_A = 125
J_A = (E // NS) // K_A
NBUF = 5

K_D = 125
J_D = (E // NW) // K_D
DLANE = 16

RPT = 640
RPT_LAST = N - (NS - 1) * RPT

_BLK = 2000

_mesh = plsc.VectorSubcoreMesh(core_axis_name="c", subcore_axis_name="s")
_sc_params = pltpu.CompilerParams(use_tc_tiling_on_sc=False)


def _slice_copy(s, src_ref, dst_ref, dst_off):
    @pl.when(s < NS - 1)
    def _():
        row0 = s * RPT
        pltpu.sync_copy(src_ref.at[pl.ds(row0, RPT)],
                        dst_ref.at[pl.ds(dst_off + row0, RPT)])

    @pl.when(s == NS - 1)
    def _():
        row0 = (NS - 1) * RPT
        pltpu.sync_copy(src_ref.at[pl.ds(row0, RPT_LAST)],
                        dst_ref.at[pl.ds(dst_off + row0, RPT_LAST)])


@functools.partial(
    pl.kernel,
    out_type=jax.ShapeDtypeStruct((NC * N, DLANE), jnp.float32),
    mesh=_mesh,
    scratch_types=[
        pltpu.VMEM((J_D, K_D), jnp.int32),
        pltpu.VMEM((K_D, DLANE), jnp.float32),
        pltpu.VMEM_SHARED((N, DLANE), jnp.float32),
        pltpu.SemaphoreType.DMA,
    ],
    compiler_params=_sc_params,
)
def _sc_degree(dst_hbm, zeros_hbm, out_hbm, dst_v, ones_v, acc_sh, sem):
    c = lax.axis_index("c")
    s = lax.axis_index("s")
    wid = c * NS + s
    pltpu.sync_copy(dst_hbm.at[wid], dst_v)

    @pl.loop(0, K_D)
    def _(k):
        ones_v[k, :] = jnp.full((DLANE,), 1.0, jnp.float32)

    _slice_copy(s, zeros_hbm, acc_sh, 0)
    plsc.subcore_barrier()

    @pl.loop(0, J_D, step=8)
    def _(j0):
        copies = [
            pltpu.async_copy(ones_v, acc_sh.at[dst_v.at[j0 + u]], sem, add=True)
            for u in range(8)
        ]
        for cp in copies:
            cp.wait()

    plsc.subcore_barrier()
    _slice_copy(s, acc_sh, out_hbm, c * N)


@functools.partial(
    pl.kernel,
    out_type=jax.ShapeDtypeStruct((NC * N, HH), jnp.float32),
    mesh=_mesh,
    scratch_types=[
        pltpu.VMEM((J_A, K_A), jnp.int32),
        pltpu.VMEM((J_A, K_A), jnp.int32),
        pltpu.VMEM((NBUF, K_A, HH), jnp.float32),
        pltpu.VMEM_SHARED((N, HH), jnp.float32),
        pltpu.SemaphoreType.DMA,
    ],
    compiler_params=_sc_params,
)
def _sc_aggregate(h_hbm, src_hbm, dst_hbm, zeros_hbm, out_hbm,
                  src_v, dst_v, rows_v, acc_sh, gsem):
    c = lax.axis_index("c")
    s = lax.axis_index("s")
    wid = c * NS + s
    pltpu.sync_copy(src_hbm.at[wid], src_v)
    pltpu.sync_copy(dst_hbm.at[s], dst_v)

    for u in range(NBUF - 1):
        pltpu.async_copy(h_hbm.at[src_v.at[u]], rows_v.at[u], gsem)

    _slice_copy(s, zeros_hbm, acc_sh, 0)
    plsc.subcore_barrier()

    @pl.loop(0, J_A, step=NBUF)
    def _(j0):
        for u in range(NBUF):
            j = j0 + u
            pltpu.make_async_copy(
                h_hbm.at[src_v.at[j]], rows_v.at[u], gsem).wait()
            pltpu.sync_copy(rows_v.at[u], acc_sh.at[dst_v.at[j]], add=True)
            nxt = j + NBUF - 1

            @pl.when(nxt < J_A)
            def _():
                pltpu.async_copy(h_hbm.at[src_v.at[nxt]],
                                 rows_v.at[(u + NBUF - 1) % NBUF], gsem)

    plsc.subcore_barrier()
    _slice_copy(s, acc_sh, out_hbm, c * N)


def _dinv_block(deg_ref):
    deg = deg_ref[0, :, 0:1] + deg_ref[1, :, 0:1] + 1.0
    return lax.rsqrt(deg)


def _split(a):
    return jnp.stack([a[:, :HH], a[:, HH:]], axis=0)


def _cat(p):
    return jnp.concatenate([p[0], p[1]], axis=1)


def _mm_body(x_ref, w_ref, o_ref):
    o_ref[...] = jnp.dot(x_ref[...], w_ref[...],
                         preferred_element_type=jnp.float32)


def _tc_matmul(x, w):
    return pl.pallas_call(
        _mm_body,
        grid=(N // _BLK,),
        in_specs=[pl.BlockSpec((_BLK, H), lambda i: (i, 0)),
                  pl.BlockSpec((H, H), lambda i: (0, 0))],
        out_specs=pl.BlockSpec((_BLK, H), lambda i: (i, 0)),
        out_shape=jax.ShapeDtypeStruct((N, H), jnp.float32),
    )(x, w)


def _scale_body(h_ref, deg_ref, o_ref):
    o_ref[...] = _split(h_ref[...] * _dinv_block(deg_ref))


def _tc_scale(h, degp):
    return pl.pallas_call(
        _scale_body,
        grid=(N // _BLK,),
        in_specs=[pl.BlockSpec((_BLK, H), lambda i: (i, 0)),
                  pl.BlockSpec((2, _BLK, DLANE), lambda i: (0, i, 0))],
        out_specs=pl.BlockSpec((2, _BLK, HH), lambda i: (0, i, 0)),
        out_shape=jax.ShapeDtypeStruct((2, N, HH), jnp.float32),
    )(h, degp)


def _mid_body(p_ref, hp_ref, deg_ref, b_ref, w_ref, o_ref):
    dinv = _dinv_block(deg_ref)
    t = (_cat(p_ref[...]) + _cat(hp_ref[...])) * dinv + b_ref[...]
    t = jnp.maximum(t, 0.0)
    o_ref[...] = _split(
        jnp.dot(t, w_ref[...], preferred_element_type=jnp.float32) * dinv)


def _tc_mid(p, hp, degp, b, w):
    return pl.pallas_call(
        _mid_body,
        grid=(N // _BLK,),
        in_specs=[pl.BlockSpec((2, _BLK, HH), lambda i: (0, i, 0)),
                  pl.BlockSpec((2, _BLK, HH), lambda i: (0, i, 0)),
                  pl.BlockSpec((2, _BLK, DLANE), lambda i: (0, i, 0)),
                  pl.BlockSpec((1, H), lambda i: (0, 0)),
                  pl.BlockSpec((H, H), lambda i: (0, 0))],
        out_specs=pl.BlockSpec((2, _BLK, HH), lambda i: (0, i, 0)),
        out_shape=jax.ShapeDtypeStruct((2, N, HH), jnp.float32),
    )(p, hp, degp, b, w)


def _final_body(p_ref, hp_ref, deg_ref, b_ref, batch_ref, wc_ref, bc_ref,
                o_ref, sums_ref, cnt_ref):
    i = pl.program_id(0)

    @pl.when(i == 0)
    def _():
        sums_ref[...] = jnp.zeros_like(sums_ref)
        cnt_ref[...] = jnp.zeros_like(cnt_ref)

    dinv = _dinv_block(deg_ref)
    t = (_cat(p_ref[...]) + _cat(hp_ref[...])) * dinv + b_ref[...]
    x4 = jnp.maximum(t, 0.0)
    bvec = batch_ref[0, 0, :]
    gids = lax.broadcasted_iota(jnp.int32, (G, _BLK), 0)
    sel = (gids == bvec[None, :]).astype(jnp.float32)
    sums_ref[...] += jnp.dot(sel, x4, preferred_element_type=jnp.float32)
    cnt_ref[...] += jnp.broadcast_to(
        jnp.sum(sel, axis=1, keepdims=True), cnt_ref.shape)

    @pl.when(i == pl.num_programs(0) - 1)
    def _():
        pooled = sums_ref[...] / jnp.maximum(cnt_ref[...], 1.0)
        o_ref[...] = jnp.dot(pooled, wc_ref[...],
                             preferred_element_type=jnp.float32) + bc_ref[...]


def _tc_final(p, hp, degp, b, batch3, wc, bcr):
    return pl.pallas_call(
        _final_body,
        grid=(N // _BLK,),
        in_specs=[pl.BlockSpec((2, _BLK, HH), lambda i: (0, i, 0)),
                  pl.BlockSpec((2, _BLK, HH), lambda i: (0, i, 0)),
                  pl.BlockSpec((2, _BLK, DLANE), lambda i: (0, i, 0)),
                  pl.BlockSpec((1, H), lambda i: (0, 0)),
                  pl.BlockSpec((1, 1, _BLK), lambda i: (i, 0, 0)),
                  pl.BlockSpec((H, C), lambda i: (0, 0)),
                  pl.BlockSpec((1, C), lambda i: (0, 0))],
        out_specs=pl.BlockSpec((G, C), lambda i: (0, 0)),
        out_shape=jax.ShapeDtypeStruct((G, C), jnp.float32),
        scratch_shapes=[pltpu.VMEM((G, H), jnp.float32),
                        pltpu.VMEM((G, H), jnp.float32)],
    )(p, hp, degp, b, batch3, wc, bcr)


def kernel(x, edge_index, batch, W1, b1, W2, b2, W3, b3, Wc, bc):
    src_r = edge_index[0].reshape(NS, J_A, K_A)
    src2 = jnp.concatenate([src_r, src_r + N], axis=0)
    dst2 = edge_index[1].reshape(NS, J_A, K_A)
    dstd = edge_index[1].reshape(NW, J_D, K_D)

    zeros_h = jnp.zeros((N, HH), jnp.float32)
    zeros_d = jnp.zeros((N, DLANE), jnp.float32)
    batch3 = batch.reshape(N // _BLK, 1, _BLK)
    b1r, b2r, b3r = b1.reshape(1, H), b2.reshape(1, H), b3.reshape(1, H)
    bcr = bc.reshape(1, C)

    degp = _sc_degree(dstd, zeros_d).reshape(NC, N, DLANE)
    h1 = _tc_matmul(x, W1)
    h1p = _tc_scale(h1, degp)
    p1 = _sc_aggregate(h1p.reshape(NC * N, HH), src2, dst2,
                       zeros_h).reshape(NC, N, HH)
    h2p = _tc_mid(p1, h1p, degp, b1r, W2)
    p2 = _sc_aggregate(h2p.reshape(NC * N, HH), src2, dst2,
                       zeros_h).reshape(NC, N, HH)
    h3p = _tc_mid(p2, h2p, degp, b2r, W3)
    p3 = _sc_aggregate(h3p.reshape(NC * N, HH), src2, dst2,
                       zeros_h).reshape(NC, N, HH)
    return _tc_final(p3, h3p, degp, b3r, batch3, Wc, bcr)
